# baseline (speedup 1.0000x reference)
import jax
import jax.numpy as jnp
from jax.experimental import pallas as pl
from jax.experimental.pallas import tpu as pltpu

D_MODEL = 2048
NUM_EXPERTS = 64
NUM_TOKENS = 16384
LANE = 128
CM = 1024
NBUF = 4
NCHUNK = NUM_TOKENS // CM
NROUND = NCHUNK // NBUF


def _router_body(x_hbm, wt_ref, b_ref, t_ref, w_ref, xbuf, sems):
    wt = wt_ref[...].astype(jnp.bfloat16)
    bias = b_ref[...]

    def _copy(j, s):
        return pltpu.make_async_copy(
            x_hbm.at[pl.ds(j * CM, CM), :], xbuf.at[s], sems.at[s])

    for s in range(NBUF):
        _copy(s, s).start(priority=s % 2)

    def round_fn(r, carry):
        base = r * NBUF
        for s in range(NBUF):
            j = base + s
            _copy(j, s).wait()
            logits = jax.lax.dot_general(
                xbuf[s].astype(jnp.bfloat16), wt,
                dimension_numbers=(((1,), (0,)), ((), ())),
                preferred_element_type=jnp.float32,
            ) + bias
            m = jnp.max(logits, axis=-1, keepdims=True)
            e = jnp.exp(logits - m)
            ssum = jnp.sum(e, axis=-1, keepdims=True)
            w = e / ssum
            w_ref[pl.ds(j * CM, CM), :] = w
            t = jnp.argmax(w, axis=-1).astype(jnp.int32)
            t_ref[pl.ds(j * (CM // LANE), CM // LANE), :] = t.reshape(
                CM // LANE, LANE)
            nxt = j + NBUF

            @pl.when(nxt < NCHUNK)
            def _():
                _copy(nxt, s).start(priority=s % 2)
        return carry

    jax.lax.fori_loop(0, NROUND, round_fn, 0)


def kernel(x, W, b):
    wt = W.T
    b2 = b.reshape(1, NUM_EXPERTS)
    top1, weights = pl.pallas_call(
        _router_body,
        in_specs=[
            pl.BlockSpec(memory_space=pltpu.MemorySpace.HBM),
            pl.BlockSpec(memory_space=pltpu.MemorySpace.VMEM),
            pl.BlockSpec(memory_space=pltpu.MemorySpace.VMEM),
        ],
        out_specs=[
            pl.BlockSpec(memory_space=pltpu.MemorySpace.VMEM),
            pl.BlockSpec(memory_space=pltpu.MemorySpace.VMEM),
        ],
        out_shape=[
            jax.ShapeDtypeStruct((NUM_TOKENS // LANE, LANE), jnp.int32),
            jax.ShapeDtypeStruct((NUM_TOKENS, NUM_EXPERTS), jnp.float32),
        ],
        scratch_shapes=[
            pltpu.VMEM((NBUF, CM, D_MODEL), jnp.float32),
            pltpu.SemaphoreType.DMA((NBUF,)),
        ],
    )(x, wt, b2)
    return top1.reshape(NUM_TOKENS), weights
